# baseline (speedup 1.0000x reference)
import jax
import jax.numpy as jnp
from jax.experimental import pallas as pl

_BLK = 8192


def _body(inp_ref, w_ref, b_ref, out_ref):
    acc = jax.lax.dot_general(
        inp_ref[...], w_ref[...],
        dimension_numbers=(((1,), (1,)), ((), ())),
        preferred_element_type=jnp.float32,
    )
    out_ref[...] = acc + b_ref[...]


def kernel(input, W, b):
    out_dim, rep = W.shape
    bsz = input.shape[0]
    grid = pl.cdiv(out_dim, _BLK)
    b2 = b.reshape(1, out_dim)
    return pl.pallas_call(
        _body,
        grid=(grid,),
        in_specs=[
            pl.BlockSpec((bsz, rep), lambda i: (0, 0)),
            pl.BlockSpec((_BLK, rep), lambda i: (i, 0)),
            pl.BlockSpec((1, _BLK), lambda i: (0, i)),
        ],
        out_specs=pl.BlockSpec((bsz, _BLK), lambda i: (0, i)),
        out_shape=jax.ShapeDtypeStruct((bsz, out_dim), jnp.float32),
    )(input, W, b2)


# speedup vs baseline: 1.0478x; 1.0478x over previous
import jax
import jax.numpy as jnp
from jax.experimental import pallas as pl

_BLK = 32768


def _body(inp_ref, w_ref, b_ref, out_ref):
    acc = jax.lax.dot_general(
        inp_ref[...], w_ref[...],
        dimension_numbers=(((1,), (1,)), ((), ())),
        preferred_element_type=jnp.float32,
    )
    out_ref[...] = acc + b_ref[...]


def kernel(input, W, b):
    out_dim, rep = W.shape
    bsz = input.shape[0]
    grid = pl.cdiv(out_dim, _BLK)
    b2 = b.reshape(1, out_dim)
    return pl.pallas_call(
        _body,
        grid=(grid,),
        in_specs=[
            pl.BlockSpec((bsz, rep), lambda i: (0, 0)),
            pl.BlockSpec((_BLK, rep), lambda i: (i, 0)),
            pl.BlockSpec((1, _BLK), lambda i: (0, i)),
        ],
        out_specs=pl.BlockSpec((bsz, _BLK), lambda i: (0, i)),
        out_shape=jax.ShapeDtypeStruct((bsz, out_dim), jnp.float32),
    )(input, W, b2)


# speedup vs baseline: 5.7276x; 5.4663x over previous
import jax
import jax.numpy as jnp
from jax.experimental import pallas as pl

_BLK = 16384


def _body(inp_ref, wt_ref, b_ref, out_ref):
    acc = jax.lax.dot_general(
        inp_ref[...], wt_ref[...],
        dimension_numbers=(((1,), (0,)), ((), ())),
        preferred_element_type=jnp.float32,
    )
    out_ref[...] = acc + b_ref[...][None, :]


def kernel(input, W, b):
    out_dim, rep = W.shape
    bsz = input.shape[0]
    wt = W.T
    grid = pl.cdiv(out_dim, _BLK)
    return pl.pallas_call(
        _body,
        grid=(grid,),
        in_specs=[
            pl.BlockSpec((bsz, rep), lambda i: (0, 0)),
            pl.BlockSpec((rep, _BLK), lambda i: (0, i)),
            pl.BlockSpec((_BLK,), lambda i: (i,)),
        ],
        out_specs=pl.BlockSpec((bsz, _BLK), lambda i: (0, i)),
        out_shape=jax.ShapeDtypeStruct((bsz, out_dim), jnp.float32),
    )(input, wt, b)


# speedup vs baseline: 6.2240x; 1.0867x over previous
import jax
import jax.numpy as jnp
from jax.experimental import pallas as pl

_BLK = 32768


def _body(inp_ref, wt_ref, b_ref, out_ref):
    acc = jax.lax.dot_general(
        inp_ref[...], wt_ref[...],
        dimension_numbers=(((1,), (0,)), ((), ())),
        preferred_element_type=jnp.float32,
    )
    out_ref[...] = acc + b_ref[...][None, :]


def kernel(input, W, b):
    out_dim, rep = W.shape
    bsz = input.shape[0]
    wt = W.T
    grid = pl.cdiv(out_dim, _BLK)
    return pl.pallas_call(
        _body,
        grid=(grid,),
        in_specs=[
            pl.BlockSpec((bsz, rep), lambda i: (0, 0)),
            pl.BlockSpec((rep, _BLK), lambda i: (0, i)),
            pl.BlockSpec((_BLK,), lambda i: (i,)),
        ],
        out_specs=pl.BlockSpec((bsz, _BLK), lambda i: (0, i)),
        out_shape=jax.ShapeDtypeStruct((bsz, out_dim), jnp.float32),
    )(input, wt, b)
